# baseline (speedup 1.0000x reference)
import dataclasses

import jax
import jax.numpy as jnp
from jax import lax
from jax.experimental import pallas as pl
from jax.experimental.pallas import tpu as pltpu
from jax.experimental.pallas import tpu_sc as plsc

N_NODES = 10000
N_EDGES = 320000
D = 128

NC = 2
NS = 16
NW = NC * NS
L = 16
CHUNK = 64
NCHUNK = N_EDGES // CHUNK
SPAN = 160
NCHUNK_PAD = NW * SPAN
BE = 2000

_HIGHEST = jax.lax.Precision.HIGHEST


def _silu(x):
    return x * jax.nn.sigmoid(x)


def _tables_body(x_ref, w1a_ref, w1b_ref, ts_ref, td_ref):
    x = x_ref[...]
    ts_ref[...] = jnp.dot(x, w1a_ref[...], preferred_element_type=jnp.float32,
                          precision=_HIGHEST)
    td_ref[...] = jnp.dot(x, w1b_ref[...], preferred_element_type=jnp.float32,
                          precision=_HIGHEST)


def _make_tables(X, w1a, w1b):
    bn = 1000
    return pl.pallas_call(
        _tables_body,
        grid=(N_NODES // bn,),
        in_specs=[
            pl.BlockSpec((bn, D), lambda i: (i, 0)),
            pl.BlockSpec((D, D), lambda i: (0, 0)),
            pl.BlockSpec((D, D), lambda i: (0, 0)),
        ],
        out_specs=[
            pl.BlockSpec((bn, D), lambda i: (i, 0)),
            pl.BlockSpec((bn, D), lambda i: (i, 0)),
        ],
        out_shape=[
            jax.ShapeDtypeStruct((N_NODES, D), jnp.float32),
            jax.ShapeDtypeStruct((N_NODES, D), jnp.float32),
        ],
    )(X, w1a, w1b)


HSPAN = SPAN // 2


def _nrm16(emb_v, i16s, i16d):
    i4s = i16s * 4
    i4d = i16d * 4
    n2 = None
    for comp in range(3):
        es = plsc.load_gather(emb_v, [i4s + comp])
        ed = plsc.load_gather(emb_v, [i4d + comp])
        dd = ed - es
        n2 = dd * dd if n2 is None else n2 + dd * dd
    n2c = jnp.maximum(n2, jnp.float32(1e-30))
    i = plsc.bitcast(n2c, jnp.int32)
    i = jnp.int32(0x5F3759DF) - jax.lax.shift_right_logical(i, 1)
    y = plsc.bitcast(i, jnp.float32)
    for _ in range(3):
        y = y * (jnp.float32(1.5) - jnp.float32(0.5) * n2c * y * y)
    return n2 * y


def _gather_body(ts_hbm, td_hbm, src_hbm, dst_hbm, ep_hbm,
                 gs_hbm, gd_hbm, nrm_hbm,
                 idx_s, idx_d, rows_s0, rows_s1, rows_d0, rows_d1,
                 emb_v, nrm_v, sem_s0, sem_s1, sem_d0, sem_d1):
    wid = lax.axis_index("s") * NC + lax.axis_index("c")
    pltpu.sync_copy(ep_hbm, emb_v)
    iota16 = jax.lax.iota(jnp.int32, L)
    zeros16 = jnp.full((L,), 0, jnp.int32)

    def start(j, rs, rd, ss, sd):
        pltpu.make_async_copy(ts_hbm.at[idx_s.at[j]], rs, ss).start()
        pltpu.make_async_copy(td_hbm.at[idx_d.at[j]], rd, sd).start()

    def finish(c, j, rs, rd, ss, sd):
        for k in range(CHUNK // L):
            nv = _nrm16(emb_v, idx_s[j, pl.ds(k * L, L)],
                        idx_d[j, pl.ds(k * L, L)])
            plsc.store_scatter(nrm_v, [iota16 + k * L, zeros16], nv)
        base = c * CHUNK
        pltpu.sync_copy(nrm_v, nrm_hbm.at[pl.ds(base, CHUNK)])
        pltpu.make_async_copy(ts_hbm.at[idx_s.at[j]], rs, ss).wait()
        pltpu.sync_copy(rs, gs_hbm.at[pl.ds(base, CHUNK)])
        pltpu.make_async_copy(td_hbm.at[idx_d.at[j]], rd, sd).wait()
        pltpu.sync_copy(rd, gd_hbm.at[pl.ds(base, CHUNK)])

    for half in range(2):
        cbase = wid * SPAN + half * HSPAN
        row0 = wid * SPAN + half * HSPAN
        pltpu.sync_copy(src_hbm.at[pl.ds(row0, HSPAN)], idx_s)
        pltpu.sync_copy(dst_hbm.at[pl.ds(row0, HSPAN)], idx_d)

        @pl.when(cbase < NCHUNK)
        def _():
            start(0, rows_s0, rows_d0, sem_s0, sem_d0)

        @pl.loop(0, HSPAN, step=2)
        def _(j):
            c0 = cbase + j
            c1 = c0 + 1

            @pl.when(c1 < NCHUNK)
            def _():
                start(j + 1, rows_s1, rows_d1, sem_s1, sem_d1)

            @pl.when(c0 < NCHUNK)
            def _():
                finish(c0, j, rows_s0, rows_d0, sem_s0, sem_d0)

            @pl.when((j + 2 < HSPAN) & (c0 + 2 < NCHUNK))
            def _():
                start(j + 2, rows_s0, rows_d0, sem_s0, sem_d0)

            @pl.when(c1 < NCHUNK)
            def _():
                finish(c1, j + 1, rows_s1, rows_d1, sem_s1, sem_d1)


def _sc_compiler_params():
    cp = pltpu.CompilerParams()
    if "needs_layout_passes" in pltpu.CompilerParams.__dataclass_fields__:
        cp = dataclasses.replace(cp, needs_layout_passes=False)
    return cp


def _sc_gather(ts, td, src2d, dst2d, ep4):
    mesh = plsc.VectorSubcoreMesh(core_axis_name="c", subcore_axis_name="s",
                                  num_cores=NC, num_subcores=NS)
    kern = pl.kernel(
        _gather_body,
        compiler_params=_sc_compiler_params(),
        out_type=[
            jax.ShapeDtypeStruct((N_EDGES, D), jnp.float32),
            jax.ShapeDtypeStruct((N_EDGES, D), jnp.float32),
            jax.ShapeDtypeStruct((N_EDGES, 16), jnp.float32),
        ],
        mesh=mesh,
        scratch_types=[
            pltpu.VMEM((HSPAN, CHUNK), jnp.int32),
            pltpu.VMEM((HSPAN, CHUNK), jnp.int32),
            pltpu.VMEM((CHUNK, D), jnp.float32),
            pltpu.VMEM((CHUNK, D), jnp.float32),
            pltpu.VMEM((CHUNK, D), jnp.float32),
            pltpu.VMEM((CHUNK, D), jnp.float32),
            pltpu.VMEM((4 * N_NODES,), jnp.float32),
            pltpu.VMEM((CHUNK, 16), jnp.float32),
            pltpu.SemaphoreType.DMA,
            pltpu.SemaphoreType.DMA,
            pltpu.SemaphoreType.DMA,
            pltpu.SemaphoreType.DMA,
        ],
    )
    return kern(ts, td, src2d, dst2d, ep4)


def _edge_body(gs_ref, gd_ref, nrm_ref, ee_ref, wn16_ref, wd8_ref, b1_ref,
               we2_ref, b2_ref, out_ref):
    pre1 = (gs_ref[...] + gd_ref[...] + b1_ref[...]
            + jnp.dot(nrm_ref[...], wn16_ref[...],
                      preferred_element_type=jnp.float32)
            + jnp.dot(ee_ref[...], wd8_ref[...],
                      preferred_element_type=jnp.float32))
    t = _silu(pre1)
    pre2 = jnp.dot(t, we2_ref[...], preferred_element_type=jnp.float32,
                   precision=_HIGHEST) + b2_ref[...]
    out_ref[...] = _silu(pre2)


def _edge_mlp(gs, gd, nrm, ee8, wn16, wd8, b1, we2, b2):
    return pl.pallas_call(
        _edge_body,
        grid=(N_EDGES // BE,),
        in_specs=[
            pl.BlockSpec((BE, D), lambda i: (i, 0)),
            pl.BlockSpec((BE, D), lambda i: (i, 0)),
            pl.BlockSpec((BE, 16), lambda i: (i, 0)),
            pl.BlockSpec((BE, 8), lambda i: (i, 0)),
            pl.BlockSpec((16, D), lambda i: (0, 0)),
            pl.BlockSpec((8, D), lambda i: (0, 0)),
            pl.BlockSpec((1, D), lambda i: (0, 0)),
            pl.BlockSpec((D, D), lambda i: (0, 0)),
            pl.BlockSpec((1, D), lambda i: (0, 0)),
        ],
        out_specs=pl.BlockSpec((BE, D), lambda i: (i, 0)),
        out_shape=jax.ShapeDtypeStruct((N_EDGES, D), jnp.float32),
    )(gs, gd, nrm, ee8, wn16, wd8, b1, we2, b2)


ACC_ROWS = N_NODES


def _scatter_body(mij_hbm, dst_hbm, zeros_hbm, p_hbm, idx_v, rows_v0, rows_v1,
                  acc, sem0, sem1):
    cid = lax.axis_index("c")
    sid = lax.axis_index("s")

    @pl.when(sid == 0)
    def _():
        pltpu.sync_copy(zeros_hbm, acc)

    wid = sid * NC + cid
    plsc.subcore_barrier()

    def start(c, rv, sem):
        pltpu.make_async_copy(mij_hbm.at[pl.ds(c * CHUNK, CHUNK)], rv,
                              sem).start()

    def finish(c, j, rv, sem):
        pltpu.make_async_copy(mij_hbm.at[pl.ds(c * CHUNK, CHUNK)], rv,
                              sem).wait()
        pltpu.sync_copy(rv, acc.at[idx_v.at[j]], add=True)

    for half in range(2):
        cbase = wid * SPAN + half * HSPAN
        pltpu.sync_copy(dst_hbm.at[pl.ds(cbase, HSPAN)], idx_v)

        @pl.when(cbase < NCHUNK)
        def _():
            start(cbase, rows_v0, sem0)

        @pl.loop(0, HSPAN, step=2)
        def _(j):
            c0 = cbase + j
            c1 = c0 + 1

            @pl.when(c1 < NCHUNK)
            def _():
                start(c1, rows_v1, sem1)

            @pl.when(c0 < NCHUNK)
            def _():
                finish(c0, j, rows_v0, sem0)

            @pl.when((j + 2 < HSPAN) & (c0 + 2 < NCHUNK))
            def _():
                start(c0 + 2, rows_v0, sem0)

            @pl.when(c1 < NCHUNK)
            def _():
                finish(c1, j + 1, rows_v1, sem1)

    plsc.subcore_barrier()

    @pl.when(sid == 0)
    def _():
        pltpu.sync_copy(acc, p_hbm.at[cid])


def _sc_scatter(mij, dst2d, zeros):
    mesh = plsc.VectorSubcoreMesh(core_axis_name="c", subcore_axis_name="s",
                                  num_cores=NC, num_subcores=NS)
    kern = pl.kernel(
        _scatter_body,
        out_type=jax.ShapeDtypeStruct((NC, N_NODES, D), jnp.float32),
        mesh=mesh,
        compiler_params=_sc_compiler_params(),
        scratch_types=[
            pltpu.VMEM((HSPAN, CHUNK), jnp.int32),
            pltpu.VMEM((CHUNK, D), jnp.float32),
            pltpu.VMEM((CHUNK, D), jnp.float32),
            pltpu.VMEM_SHARED((ACC_ROWS, D), jnp.float32),
            pltpu.SemaphoreType.DMA,
            pltpu.SemaphoreType.DMA,
        ],
    )
    return kern(mij, dst2d, zeros)


def _node_body(x_ref, p0_ref, p1_ref, w1x_ref, w1m_ref, b1_ref, w2_ref,
               b2_ref, out_ref):
    x = x_ref[...]
    mi = p0_ref[...] + p1_ref[...]
    t = _silu(jnp.dot(x, w1x_ref[...], preferred_element_type=jnp.float32,
                      precision=_HIGHEST)
              + jnp.dot(mi, w1m_ref[...], preferred_element_type=jnp.float32,
                        precision=_HIGHEST)
              + b1_ref[...])
    out_ref[...] = jnp.dot(t, w2_ref[...], preferred_element_type=jnp.float32,
                           precision=_HIGHEST) + b2_ref[...]


def _node_mlp(X, p0, p1, w1x, w1m, b1, w2, b2):
    bn = 1000
    return pl.pallas_call(
        _node_body,
        grid=(N_NODES // bn,),
        in_specs=[
            pl.BlockSpec((bn, D), lambda i: (i, 0)),
            pl.BlockSpec((bn, D), lambda i: (i, 0)),
            pl.BlockSpec((bn, D), lambda i: (i, 0)),
            pl.BlockSpec((D, D), lambda i: (0, 0)),
            pl.BlockSpec((D, D), lambda i: (0, 0)),
            pl.BlockSpec((1, D), lambda i: (0, 0)),
            pl.BlockSpec((D, D), lambda i: (0, 0)),
            pl.BlockSpec((1, D), lambda i: (0, 0)),
        ],
        out_specs=pl.BlockSpec((bn, D), lambda i: (i, 0)),
        out_shape=jax.ShapeDtypeStruct((N_NODES, D), jnp.float32),
    )(X, p0, p1, w1x, w1m, b1, w2, b2)


def kernel(X, E, emb_nodes, emb_edges, edge_index,
           W_e1, b_e1, W_e2, b_e2, W_h1, b_h1, W_h2, b_h2):
    src = edge_index[0]
    dst = edge_index[1]
    pad_rows = NCHUNK_PAD - NCHUNK
    src2d = jnp.pad(src.reshape(NCHUNK, CHUNK), ((0, pad_rows), (0, 0)))
    dst2d = jnp.pad(dst.reshape(NCHUNK, CHUNK), ((0, pad_rows), (0, 0)))
    ep4 = jnp.pad(emb_nodes, ((0, 0), (0, 4 - emb_nodes.shape[1]))).reshape(-1)
    ee8 = jnp.pad(emb_edges, ((0, 0), (0, 8 - emb_edges.shape[1])))

    w1a = W_e1[0:D]
    w1b = W_e1[D:2 * D]
    wn16 = jnp.pad(W_e1[2 * D:2 * D + 1], ((0, 15), (0, 0)))
    wd8 = jnp.pad(W_e1[2 * D + 1:], ((0, 6), (0, 0)))
    b1 = b_e1.reshape(1, D)
    b2 = b_e2.reshape(1, D)

    ts, td = _make_tables(X, w1a, w1b)
    gs, gd, nrm = _sc_gather(ts, td, src2d, dst2d, ep4)
    mij = _edge_mlp(gs, gd, nrm, ee8, wn16, wd8, b1, W_e2, b2)
    zeros = jnp.zeros((ACC_ROWS, D), jnp.float32)
    parts = _sc_scatter(mij, dst2d, zeros)
    X_out = _node_mlp(X, parts[0], parts[1], W_h1[0:D], W_h1[D:],
                      b_h1.reshape(1, D), W_h2, b_h2.reshape(1, D))
    return (X_out, mij, emb_nodes, emb_edges)
